# baseline (speedup 1.0000x reference)
import jax
import jax.numpy as jnp
from jax import lax
from jax.experimental import pallas as pl
from jax.experimental.pallas import tpu as pltpu
from jax.experimental.pallas import tpu_sc as plsc

V = 10000
E = 320000
H = 128
NW = 32
C = 80
VPS = 624
VTAIL = V - 16 * VPS

EA = 163840
EB = E - EA

BE = 2560
BV = 2000

_mesh = lambda: plsc.VectorSubcoreMesh(core_axis_name="c", subcore_axis_name="s")


def _worker(base_mul):
    cid = lax.axis_index("c")
    sid = lax.axis_index("s")
    wid = sid * 2 + cid
    return cid, sid, wid, wid * base_mul


def _make_gather_body(EPWx, NCHx):
    def _gather_body(A_hbm, B_hbm, dst3_hbm, src3_hbm, G_hbm,
                     idxd, idxs, bufA0, bufA1, bufA2, bufB0, bufB1, bufB2,
                     semA0, semA1, semA2, semB0, semB1, semB2,
                     semWA0, semWA1, semWA2, semWB0, semWB1, semWB2):
        _, _, wid, base = _worker(EPWx)
        bufA = [bufA0, bufA1, bufA2]
        bufB = [bufB0, bufB1, bufB2]
        semA = [semA0, semA1, semA2]
        semB = [semB0, semB1, semB2]
        semWA = [semWA0, semWA1, semWA2]
        semWB = [semWB0, semWB1, semWB2]

        pltpu.sync_copy(dst3_hbm.at[wid], idxd)
        pltpu.sync_copy(src3_hbm.at[wid], idxs)

        def issue(j, b):
            pltpu.async_copy(A_hbm.at[idxd.at[j]], bufA[b], semA[b])
            pltpu.async_copy(B_hbm.at[idxs.at[j]], bufB[b], semB[b])

        def wait_writes(j, b):
            pltpu.make_async_copy(
                bufA[b], G_hbm.at[pl.ds(0, C), pl.ds(0, H // 2)], semWA[b]).wait()
            pltpu.make_async_copy(
                bufB[b], G_hbm.at[pl.ds(0, C), pl.ds(H // 2, H // 2)], semWB[b]).wait()

        def step(j, b):
            pltpu.make_async_copy(A_hbm.at[idxd.at[j]], bufA[b], semA[b]).wait()
            pltpu.make_async_copy(B_hbm.at[idxs.at[j]], bufB[b], semB[b]).wait()
            off = base + j * C
            pltpu.async_copy(bufA[b], G_hbm.at[pl.ds(off, C), pl.ds(0, H // 2)],
                             semWA[b])
            pltpu.async_copy(bufB[b], G_hbm.at[pl.ds(off, C), pl.ds(H // 2, H // 2)],
                             semWB[b])

            bp = (b + 2) % 3

            @pl.when(j >= 1)
            def _():
                wait_writes(j - 1, bp)

            @pl.when(j + 2 < NCHx)
            def _():
                issue(j + 2, bp)

        issue(0, 0)
        issue(1, 1)

        def triple(i, _):
            step(3 * i, 0)
            step(3 * i + 1, 1)
            step(3 * i + 2, 2)
            return _
        nt = (NCHx - 2) // 3
        lax.fori_loop(0, nt, triple, None)
        for j in range(3 * nt, NCHx):
            step(j, j % 3)
        wait_writes(NCHx - 1, (NCHx - 1) % 3)
    return _gather_body


def _sc_gather(A32, B32, dst3, src3, Ex):
    EPWx = Ex // NW
    NCHx = EPWx // C
    bufs = lambda: pltpu.VMEM((C, H // 2), jnp.int32)
    sems = lambda: pltpu.SemaphoreType.DMA
    return pl.kernel(
        _make_gather_body(EPWx, NCHx),
        out_type=jax.ShapeDtypeStruct((Ex, H), jnp.int32),
        mesh=_mesh(),
        compiler_params=pltpu.CompilerParams(use_tc_tiling_on_sc=False),
        scratch_types=[
            pltpu.VMEM((NCHx, C), jnp.int32),
            pltpu.VMEM((NCHx, C), jnp.int32),
            bufs(), bufs(), bufs(), bufs(), bufs(), bufs(),
            sems(), sems(), sems(), sems(), sems(), sems(),
            sems(), sems(), sems(), sems(), sems(), sems(),
        ],
    )(A32, B32, dst3, src3)


def _make_scatter_body(EPWx, NCHx):
    def _scatter_body(e2_hbm, dst3_hbm, part_hbm,
                      idxd, buf0, buf1, zbuf, agg_sh, semR0, semR1):
        cid, sid, wid, base = _worker(EPWx)
        buf = [buf0, buf1]
        semR = [semR0, semR1]

        pltpu.sync_copy(dst3_hbm.at[wid], idxd)

        def zrow(i, _):
            def zcol(k, __):
                zbuf[i, pl.ds(k * 16, 16)] = jnp.zeros((16,), jnp.float32)
                return __
            return lax.fori_loop(0, H // 16, zcol, _)
        lax.fori_loop(0, C, zrow, None)
        for r in range(7):
            pltpu.sync_copy(zbuf, agg_sh.at[pl.ds(sid * VPS + r * C, C)])
        pltpu.sync_copy(zbuf.at[pl.ds(0, VPS - 7 * C)],
                        agg_sh.at[pl.ds(sid * VPS + 7 * C, VPS - 7 * C)])

        @pl.when(sid == 0)
        def _():
            pltpu.sync_copy(zbuf.at[pl.ds(0, VTAIL)],
                            agg_sh.at[pl.ds(16 * VPS, VTAIL)])

        plsc.subcore_barrier()

        def issue(j, b):
            pltpu.async_copy(e2_hbm.at[pl.ds(base + j * C, C)], buf[b], semR[b])

        def step(j, b):
            pltpu.make_async_copy(e2_hbm.at[pl.ds(0, C)], buf[b], semR[b]).wait()
            pltpu.sync_copy(buf[b], agg_sh.at[idxd.at[j]], add=True)

            @pl.when(j + 2 < NCHx)
            def _():
                issue(j + 2, b)

        issue(0, 0)
        issue(1, 1)

        def pair(i, _):
            step(2 * i, 0)
            step(2 * i + 1, 1)
            return _
        lax.fori_loop(0, NCHx // 2, pair, None)
        for j in range(2 * (NCHx // 2), NCHx):
            step(j, j % 2)

        plsc.subcore_barrier()
        pltpu.sync_copy(agg_sh.at[pl.ds(sid * VPS, VPS)],
                        part_hbm.at[cid, pl.ds(sid * VPS, VPS)])

        @pl.when(sid == 0)
        def _():
            pltpu.sync_copy(agg_sh.at[pl.ds(16 * VPS, VTAIL)],
                            part_hbm.at[cid, pl.ds(16 * VPS, VTAIL)])
    return _scatter_body


def _sc_scatter(e2, dst3, Ex):
    EPWx = Ex // NW
    NCHx = EPWx // C
    return pl.kernel(
        _make_scatter_body(EPWx, NCHx),
        out_type=jax.ShapeDtypeStruct((2, V, H), jnp.float32),
        mesh=_mesh(),
        scratch_types=[
            pltpu.VMEM((NCHx, C), jnp.int32),
            pltpu.VMEM((C, H), jnp.float32),
            pltpu.VMEM((C, H), jnp.float32),
            pltpu.VMEM((C, H), jnp.float32),
            pltpu.VMEM_SHARED((V, H), jnp.float32),
            pltpu.SemaphoreType.DMA, pltpu.SemaphoreType.DMA,
        ],
    )(e2, dst3)


def _silu(x):
    return x * jax.nn.sigmoid(x)


def _pack_bf16(x):
    x = x.astype(jnp.bfloat16).astype(jnp.float32)
    return pltpu.pack_elementwise([x[:, :H // 2], x[:, H // 2:]],
                                  packed_dtype=jnp.bfloat16)


def _unpack_bf16(x32):
    lo = pltpu.unpack_elementwise(x32, index=0, packed_dtype=jnp.bfloat16,
                                  unpacked_dtype=jnp.float32)
    hi = pltpu.unpack_elementwise(x32, index=1, packed_dtype=jnp.bfloat16,
                                  unpacked_dtype=jnp.float32)
    return lo, hi


def _prep_kernel(z_ref, temb_ref, cpzT_ref, cptT_ref, bc_ref,
                 WdT_ref, WsT_ref, Wr8T_ref, ppos_ref,
                 h_ref, A_ref, B_ref):
    h0 = (jnp.dot(z_ref[...], cpzT_ref[...], preferred_element_type=jnp.float32)
          + jnp.dot(temb_ref[...], cptT_ref[...], preferred_element_type=jnp.float32)
          + bc_ref[...])
    h = jnp.broadcast_to(h0, (BV, H))
    h_ref[...] = h
    pw = jnp.dot(ppos_ref[...], Wr8T_ref[...], preferred_element_type=jnp.float32)
    hA = jnp.dot(h0, WdT_ref[...], preferred_element_type=jnp.float32)
    hB = jnp.dot(h0, WsT_ref[...], preferred_element_type=jnp.float32)
    A_ref[...] = _pack_bf16(hA + pw)
    B_ref[...] = _pack_bf16(hB - pw)


def _tc_prep(z, temb, cpzT, cptT, bc, WdT, WsT, Wr8T, ppos):
    full = lambda s: pl.BlockSpec(s, lambda i: (0,) * len(s))
    return pl.pallas_call(
        _prep_kernel,
        grid=(V // BV,),
        in_specs=[full((1, 64)), full((1, 16)), full((64, H)), full((16, H)),
                  full((1, H)), full((H, H)), full((H, H)), full((8, H)),
                  pl.BlockSpec((BV, 8), lambda i: (i, 0))],
        out_specs=[pl.BlockSpec((BV, H), lambda i: (i, 0)),
                   pl.BlockSpec((BV, H // 2), lambda i: (i, 0)),
                   pl.BlockSpec((BV, H // 2), lambda i: (i, 0))],
        out_shape=[jax.ShapeDtypeStruct((V, H), jnp.float32),
                   jax.ShapeDtypeStruct((V, H // 2), jnp.int32),
                   jax.ShapeDtypeStruct((V, H // 2), jnp.int32)],
    )(z, temb, cpzT, cptT, bc, WdT, WsT, Wr8T, ppos)


def _edge_kernel(G_ref, b1_ref, W2T_ref, b2_ref, e2_ref):
    g = G_ref[...]
    alo, ahi = _unpack_bf16(g[:, :H // 2])
    blo, bhi = _unpack_bf16(g[:, H // 2:])
    x = jnp.concatenate([alo + blo, ahi + bhi], axis=-1) + b1_ref[...]
    e1 = _silu(x).astype(jnp.bfloat16)
    e2_ref[...] = _silu(
        jnp.dot(e1, W2T_ref[...], preferred_element_type=jnp.float32) + b2_ref[...])


def _tc_edge(G32, b1, W2T, b2, Ex):
    full = lambda s: pl.BlockSpec(s, lambda i: (0,) * len(s))
    return pl.pallas_call(
        _edge_kernel,
        grid=(Ex // BE,),
        in_specs=[pl.BlockSpec((BE, H), lambda i: (i, 0)),
                  full((1, H)), full((H, H)), full((1, H))],
        out_specs=pl.BlockSpec((BE, H), lambda i: (i, 0)),
        out_shape=jax.ShapeDtypeStruct((Ex, H), jnp.float32),
    )(G32, b1, W2T, b2)


def _node_common(h_ref, partA_ref, partB_ref, WhT_ref, WaT_ref, bn1_ref,
                 Wn2T_ref, bn2_ref, g_ref, bln_ref):
    h = h_ref[...]
    agg = (partA_ref[0] + partA_ref[1]) + (partB_ref[0] + partB_ref[1])
    u = _silu(jnp.dot(h, WhT_ref[...], preferred_element_type=jnp.float32)
              + jnp.dot(agg, WaT_ref[...], preferred_element_type=jnp.float32)
              + bn1_ref[...])
    hn = jnp.dot(u, Wn2T_ref[...], preferred_element_type=jnp.float32) + bn2_ref[...]
    x = h + hn
    mu = jnp.mean(x, axis=-1, keepdims=True)
    r = x - mu
    var = jnp.mean(r * r, axis=-1, keepdims=True)
    return r * jax.lax.rsqrt(var + 1e-5) * g_ref[...] + bln_ref[...]


def _node_kernel(h_ref, partA_ref, partB_ref, ppos_ref, WhT_ref, WaT_ref,
                 bn1_ref, Wn2T_ref, bn2_ref, g_ref, bln_ref,
                 WdT_ref, WsT_ref, Wr8T_ref,
                 h2_ref, A_ref, B_ref):
    h2 = _node_common(h_ref, partA_ref, partB_ref, WhT_ref, WaT_ref, bn1_ref,
                      Wn2T_ref, bn2_ref, g_ref, bln_ref)
    h2_ref[...] = h2
    pw = jnp.dot(ppos_ref[...], Wr8T_ref[...], preferred_element_type=jnp.float32)
    hA = jnp.dot(h2, WdT_ref[...], preferred_element_type=jnp.float32)
    hB = jnp.dot(h2, WsT_ref[...], preferred_element_type=jnp.float32)
    A_ref[...] = _pack_bf16(hA + pw)
    B_ref[...] = _pack_bf16(hB - pw)


def _tc_node(h, partA, partB, ppos, WhT, WaT, bn1, Wn2T, bn2, g, bln,
             WdT, WsT, Wr8T):
    full = lambda s: pl.BlockSpec(s, lambda i: (0,) * len(s))
    return pl.pallas_call(
        _node_kernel,
        grid=(V // BV,),
        in_specs=[pl.BlockSpec((BV, H), lambda i: (i, 0)),
                  pl.BlockSpec((2, BV, H), lambda i: (0, i, 0)),
                  pl.BlockSpec((2, BV, H), lambda i: (0, i, 0)),
                  pl.BlockSpec((BV, 8), lambda i: (i, 0)),
                  full((H, H)), full((H, H)), full((1, H)),
                  full((H, H)), full((1, H)), full((1, H)), full((1, H)),
                  full((H, H)), full((H, H)), full((8, H))],
        out_specs=[pl.BlockSpec((BV, H), lambda i: (i, 0)),
                   pl.BlockSpec((BV, H // 2), lambda i: (i, 0)),
                   pl.BlockSpec((BV, H // 2), lambda i: (i, 0))],
        out_shape=[jax.ShapeDtypeStruct((V, H), jnp.float32),
                   jax.ShapeDtypeStruct((V, H // 2), jnp.int32),
                   jax.ShapeDtypeStruct((V, H // 2), jnp.int32)],
    )(h, partA, partB, ppos, WhT, WaT, bn1, Wn2T, bn2, g, bln, WdT, WsT, Wr8T)


def _node_final_kernel(h_ref, partA_ref, partB_ref, WhT_ref, WaT_ref, bn1_ref,
                       Wn2T_ref, bn2_ref, g_ref, bln_ref,
                       WoT_ref, bo_ref, out_ref):
    h2 = _node_common(h_ref, partA_ref, partB_ref, WhT_ref, WaT_ref, bn1_ref,
                      Wn2T_ref, bn2_ref, g_ref, bln_ref)
    out_ref[...] = jnp.dot(h2, WoT_ref[...], preferred_element_type=jnp.float32) + bo_ref[...]


def _tc_node_final(h, partA, partB, WhT, WaT, bn1, Wn2T, bn2, g, bln, WoT8, bo8):
    full = lambda s: pl.BlockSpec(s, lambda i: (0,) * len(s))
    return pl.pallas_call(
        _node_final_kernel,
        grid=(V // BV,),
        in_specs=[pl.BlockSpec((BV, H), lambda i: (i, 0)),
                  pl.BlockSpec((2, BV, H), lambda i: (0, i, 0)),
                  pl.BlockSpec((2, BV, H), lambda i: (0, i, 0)),
                  full((H, H)), full((H, H)), full((1, H)),
                  full((H, H)), full((1, H)), full((1, H)), full((1, H)),
                  full((H, 8)), full((1, 8))],
        out_specs=pl.BlockSpec((BV, 8), lambda i: (i, 0)),
        out_shape=jax.ShapeDtypeStruct((V, 8), jnp.float32),
    )(h, partA, partB, WhT, WaT, bn1, Wn2T, bn2, g, bln, WoT8, bo8)


def kernel(pos, t, z, params, edge_index, batch):
    f32 = jnp.float32
    src = edge_index[0]
    dst = edge_index[1]
    srcA = src[:EA].reshape(NW, EA // NW // C, C)
    dstA = dst[:EA].reshape(NW, EA // NW // C, C)
    srcB = src[EA:].reshape(NW, EB // NW // C, C)
    dstB = dst[EA:].reshape(NW, EB // NW // C, C)

    te0, te1 = params["time_embed"]
    temb = _silu(t[:1, None] * te0["W"][:, 0][None, :] + te0["b"][None, :])
    temb = temb @ te1["W"].T + te1["b"][None, :]

    cw = params["cond_proj"]["W"]
    cpzT = jnp.asarray(cw[:, :64].T, f32)
    cptT = jnp.asarray(cw[:, 64:].T, f32)
    bc = params["cond_proj"]["b"][None, :]

    ppos = jnp.pad(pos, ((0, 0), (0, 5)))

    def split_w1(lp):
        W1 = lp["edge_mlp"][0]["W"]
        WdT = W1[:, :H].T
        WsT = W1[:, H:2 * H].T
        Wr8T = jnp.pad(W1[:, 2 * H:].T, ((0, 5), (0, 0)))
        return WdT, WsT, Wr8T

    lps = params["layers"]
    WdT0, WsT0, Wr8T0 = split_w1(lps[0])
    h, A, B = _tc_prep(z, temb, cpzT, cptT, bc, WdT0, WsT0, Wr8T0, ppos)

    for li, lp in enumerate(lps):
        b1 = lp["edge_mlp"][0]["b"][None, :]
        W2T = lp["edge_mlp"][1]["W"].T.astype(jnp.bfloat16)
        b2 = lp["edge_mlp"][1]["b"][None, :]
        GA = _sc_gather(A, B, dstA, srcA, EA)
        GB = _sc_gather(A, B, dstB, srcB, EB)
        e2A = _tc_edge(GA, b1, W2T, b2, EA)
        e2B = _tc_edge(GB, b1, W2T, b2, EB)
        partA = _sc_scatter(e2A, dstA, EA)
        partB = _sc_scatter(e2B, dstB, EB)

        n1, n2 = lp["node_mlp"]
        WhT = n1["W"][:, :H].T
        WaT = n1["W"][:, H:].T
        bn1 = n1["b"][None, :]
        Wn2T = n2["W"].T
        bn2 = n2["b"][None, :]
        g = lp["ln"]["g"][None, :]
        bln = lp["ln"]["b"][None, :]
        if li + 1 < len(lps):
            WdT, WsT, Wr8T = split_w1(lps[li + 1])
            h, A, B = _tc_node(h, partA, partB, ppos, WhT, WaT, bn1, Wn2T,
                               bn2, g, bln, WdT, WsT, Wr8T)
        else:
            WoT8 = jnp.pad(params["out_proj"]["W"].T, ((0, 0), (0, 5)))
            bo8 = jnp.pad(params["out_proj"]["b"], (0, 5))[None, :]
            out8 = _tc_node_final(h, partA, partB, WhT, WaT, bn1, Wn2T, bn2,
                                  g, bln, WoT8, bo8)
    return out8[:, :3]


# speedup vs baseline: 1.0600x; 1.0600x over previous
import jax
import jax.numpy as jnp
from jax import lax
from jax.experimental import pallas as pl
from jax.experimental.pallas import tpu as pltpu
from jax.experimental.pallas import tpu_sc as plsc

V = 10000
E = 320000
H = 128
NW = 32
C = 80
VPS = 624
VTAIL = V - 16 * VPS

ESPLIT = (107520, 107520, 104960)

BE = 2560
BV = 2000

_mesh = lambda: plsc.VectorSubcoreMesh(core_axis_name="c", subcore_axis_name="s")


def _worker(base_mul):
    cid = lax.axis_index("c")
    sid = lax.axis_index("s")
    wid = sid * 2 + cid
    return cid, sid, wid, wid * base_mul


def _make_gather_body(EPWx, NCHx):
    def _gather_body(A_hbm, B_hbm, dst3_hbm, src3_hbm, G_hbm,
                     idxd, idxs, bufA0, bufA1, bufA2, bufB0, bufB1, bufB2,
                     semA0, semA1, semA2, semB0, semB1, semB2,
                     semWA0, semWA1, semWA2, semWB0, semWB1, semWB2):
        _, _, wid, base = _worker(EPWx)
        bufA = [bufA0, bufA1, bufA2]
        bufB = [bufB0, bufB1, bufB2]
        semA = [semA0, semA1, semA2]
        semB = [semB0, semB1, semB2]
        semWA = [semWA0, semWA1, semWA2]
        semWB = [semWB0, semWB1, semWB2]

        pltpu.sync_copy(dst3_hbm.at[wid], idxd)
        pltpu.sync_copy(src3_hbm.at[wid], idxs)

        def issue(j, b):
            pltpu.async_copy(A_hbm.at[idxd.at[j]], bufA[b], semA[b])
            pltpu.async_copy(B_hbm.at[idxs.at[j]], bufB[b], semB[b])

        def wait_writes(j, b):
            pltpu.make_async_copy(
                bufA[b], G_hbm.at[pl.ds(0, C), pl.ds(0, H // 2)], semWA[b]).wait()
            pltpu.make_async_copy(
                bufB[b], G_hbm.at[pl.ds(0, C), pl.ds(H // 2, H // 2)], semWB[b]).wait()

        def step(j, b):
            pltpu.make_async_copy(A_hbm.at[idxd.at[j]], bufA[b], semA[b]).wait()
            pltpu.make_async_copy(B_hbm.at[idxs.at[j]], bufB[b], semB[b]).wait()
            off = base + j * C
            pltpu.async_copy(bufA[b], G_hbm.at[pl.ds(off, C), pl.ds(0, H // 2)],
                             semWA[b])
            pltpu.async_copy(bufB[b], G_hbm.at[pl.ds(off, C), pl.ds(H // 2, H // 2)],
                             semWB[b])

            bp = (b + 2) % 3

            @pl.when(j >= 1)
            def _():
                wait_writes(j - 1, bp)

            @pl.when(j + 2 < NCHx)
            def _():
                issue(j + 2, bp)

        issue(0, 0)
        issue(1, 1)

        def triple(i, _):
            step(3 * i, 0)
            step(3 * i + 1, 1)
            step(3 * i + 2, 2)
            return _
        nt = (NCHx - 2) // 3
        lax.fori_loop(0, nt, triple, None)
        for j in range(3 * nt, NCHx):
            step(j, j % 3)
        wait_writes(NCHx - 1, (NCHx - 1) % 3)
    return _gather_body


def _sc_gather(A32, B32, dst3, src3, Ex):
    EPWx = Ex // NW
    NCHx = EPWx // C
    bufs = lambda: pltpu.VMEM((C, H // 2), jnp.int32)
    sems = lambda: pltpu.SemaphoreType.DMA
    return pl.kernel(
        _make_gather_body(EPWx, NCHx),
        out_type=jax.ShapeDtypeStruct((Ex, H), jnp.int32),
        mesh=_mesh(),
        compiler_params=pltpu.CompilerParams(use_tc_tiling_on_sc=False),
        scratch_types=[
            pltpu.VMEM((NCHx, C), jnp.int32),
            pltpu.VMEM((NCHx, C), jnp.int32),
            bufs(), bufs(), bufs(), bufs(), bufs(), bufs(),
            sems(), sems(), sems(), sems(), sems(), sems(),
            sems(), sems(), sems(), sems(), sems(), sems(),
        ],
    )(A32, B32, dst3, src3)


def _make_scatter_body(EPWx, NCHx):
    def _scatter_body(e2_hbm, dst3_hbm, part_hbm,
                      idxd, buf0, buf1, zbuf, agg_sh, semR0, semR1):
        cid, sid, wid, base = _worker(EPWx)
        buf = [buf0, buf1]
        semR = [semR0, semR1]

        pltpu.sync_copy(dst3_hbm.at[wid], idxd)

        def zrow(i, _):
            def zcol(k, __):
                zbuf[i, pl.ds(k * 16, 16)] = jnp.zeros((16,), jnp.float32)
                return __
            return lax.fori_loop(0, H // 16, zcol, _)
        lax.fori_loop(0, C, zrow, None)
        for r in range(7):
            pltpu.sync_copy(zbuf, agg_sh.at[pl.ds(sid * VPS + r * C, C)])
        pltpu.sync_copy(zbuf.at[pl.ds(0, VPS - 7 * C)],
                        agg_sh.at[pl.ds(sid * VPS + 7 * C, VPS - 7 * C)])

        @pl.when(sid == 0)
        def _():
            pltpu.sync_copy(zbuf.at[pl.ds(0, VTAIL)],
                            agg_sh.at[pl.ds(16 * VPS, VTAIL)])

        plsc.subcore_barrier()

        def issue(j, b):
            pltpu.async_copy(e2_hbm.at[pl.ds(base + j * C, C)], buf[b], semR[b])

        def step(j, b):
            pltpu.make_async_copy(e2_hbm.at[pl.ds(0, C)], buf[b], semR[b]).wait()
            pltpu.sync_copy(buf[b], agg_sh.at[idxd.at[j]], add=True)

            @pl.when(j + 2 < NCHx)
            def _():
                issue(j + 2, b)

        issue(0, 0)
        issue(1, 1)

        def pair(i, _):
            step(2 * i, 0)
            step(2 * i + 1, 1)
            return _
        lax.fori_loop(0, NCHx // 2, pair, None)
        for j in range(2 * (NCHx // 2), NCHx):
            step(j, j % 2)

        plsc.subcore_barrier()
        pltpu.sync_copy(agg_sh.at[pl.ds(sid * VPS, VPS)],
                        part_hbm.at[cid, pl.ds(sid * VPS, VPS)])

        @pl.when(sid == 0)
        def _():
            pltpu.sync_copy(agg_sh.at[pl.ds(16 * VPS, VTAIL)],
                            part_hbm.at[cid, pl.ds(16 * VPS, VTAIL)])
    return _scatter_body


def _sc_scatter(e2, dst3, Ex):
    EPWx = Ex // NW
    NCHx = EPWx // C
    return pl.kernel(
        _make_scatter_body(EPWx, NCHx),
        out_type=jax.ShapeDtypeStruct((2, V, H), jnp.float32),
        mesh=_mesh(),
        scratch_types=[
            pltpu.VMEM((NCHx, C), jnp.int32),
            pltpu.VMEM((C, H), jnp.float32),
            pltpu.VMEM((C, H), jnp.float32),
            pltpu.VMEM((C, H), jnp.float32),
            pltpu.VMEM_SHARED((V, H), jnp.float32),
            pltpu.SemaphoreType.DMA, pltpu.SemaphoreType.DMA,
        ],
    )(e2, dst3)


def _silu(x):
    return x * jax.nn.sigmoid(x)


def _pack_bf16(x):
    x = x.astype(jnp.bfloat16).astype(jnp.float32)
    return pltpu.pack_elementwise([x[:, :H // 2], x[:, H // 2:]],
                                  packed_dtype=jnp.bfloat16)


def _unpack_bf16(x32):
    lo = pltpu.unpack_elementwise(x32, index=0, packed_dtype=jnp.bfloat16,
                                  unpacked_dtype=jnp.float32)
    hi = pltpu.unpack_elementwise(x32, index=1, packed_dtype=jnp.bfloat16,
                                  unpacked_dtype=jnp.float32)
    return lo, hi


def _prep_kernel(z_ref, temb_ref, cpzT_ref, cptT_ref, bc_ref,
                 WdT_ref, WsT_ref, Wr8T_ref, ppos_ref,
                 h_ref, A_ref, B_ref):
    h0 = (jnp.dot(z_ref[...], cpzT_ref[...], preferred_element_type=jnp.float32)
          + jnp.dot(temb_ref[...], cptT_ref[...], preferred_element_type=jnp.float32)
          + bc_ref[...])
    h = jnp.broadcast_to(h0, (BV, H))
    h_ref[...] = h
    pw = jnp.dot(ppos_ref[...], Wr8T_ref[...], preferred_element_type=jnp.float32)
    hA = jnp.dot(h0, WdT_ref[...], preferred_element_type=jnp.float32)
    hB = jnp.dot(h0, WsT_ref[...], preferred_element_type=jnp.float32)
    A_ref[...] = _pack_bf16(hA + pw)
    B_ref[...] = _pack_bf16(hB - pw)


def _tc_prep(z, temb, cpzT, cptT, bc, WdT, WsT, Wr8T, ppos):
    full = lambda s: pl.BlockSpec(s, lambda i: (0,) * len(s))
    return pl.pallas_call(
        _prep_kernel,
        grid=(V // BV,),
        in_specs=[full((1, 64)), full((1, 16)), full((64, H)), full((16, H)),
                  full((1, H)), full((H, H)), full((H, H)), full((8, H)),
                  pl.BlockSpec((BV, 8), lambda i: (i, 0))],
        out_specs=[pl.BlockSpec((BV, H), lambda i: (i, 0)),
                   pl.BlockSpec((BV, H // 2), lambda i: (i, 0)),
                   pl.BlockSpec((BV, H // 2), lambda i: (i, 0))],
        out_shape=[jax.ShapeDtypeStruct((V, H), jnp.float32),
                   jax.ShapeDtypeStruct((V, H // 2), jnp.int32),
                   jax.ShapeDtypeStruct((V, H // 2), jnp.int32)],
    )(z, temb, cpzT, cptT, bc, WdT, WsT, Wr8T, ppos)


def _edge_kernel(G_ref, b1_ref, W2T_ref, b2_ref, e2_ref):
    g = G_ref[...]
    alo, ahi = _unpack_bf16(g[:, :H // 2])
    blo, bhi = _unpack_bf16(g[:, H // 2:])
    x = jnp.concatenate([alo + blo, ahi + bhi], axis=-1) + b1_ref[...]
    e1 = _silu(x)
    e2_ref[...] = _silu(
        jnp.dot(e1, W2T_ref[...], preferred_element_type=jnp.float32) + b2_ref[...])


def _tc_edge(G32, b1, W2T, b2, Ex):
    full = lambda s: pl.BlockSpec(s, lambda i: (0,) * len(s))
    return pl.pallas_call(
        _edge_kernel,
        grid=(Ex // BE,),
        in_specs=[pl.BlockSpec((BE, H), lambda i: (i, 0)),
                  full((1, H)), full((H, H)), full((1, H))],
        out_specs=pl.BlockSpec((BE, H), lambda i: (i, 0)),
        out_shape=jax.ShapeDtypeStruct((Ex, H), jnp.float32),
    )(G32, b1, W2T, b2)


def _node_common(h_ref, partA_ref, partB_ref, partC_ref, WhT_ref, WaT_ref,
                 bn1_ref, Wn2T_ref, bn2_ref, g_ref, bln_ref):
    h = h_ref[...]
    agg = ((partA_ref[0] + partA_ref[1]) + (partB_ref[0] + partB_ref[1])
           + (partC_ref[0] + partC_ref[1]))
    u = _silu(jnp.dot(h, WhT_ref[...], preferred_element_type=jnp.float32)
              + jnp.dot(agg, WaT_ref[...], preferred_element_type=jnp.float32)
              + bn1_ref[...])
    hn = jnp.dot(u, Wn2T_ref[...], preferred_element_type=jnp.float32) + bn2_ref[...]
    x = h + hn
    mu = jnp.mean(x, axis=-1, keepdims=True)
    r = x - mu
    var = jnp.mean(r * r, axis=-1, keepdims=True)
    return r * jax.lax.rsqrt(var + 1e-5) * g_ref[...] + bln_ref[...]


def _node_kernel(h_ref, partA_ref, partB_ref, partC_ref, ppos_ref, WhT_ref,
                 WaT_ref, bn1_ref, Wn2T_ref, bn2_ref, g_ref, bln_ref,
                 WdT_ref, WsT_ref, Wr8T_ref,
                 h2_ref, A_ref, B_ref):
    h2 = _node_common(h_ref, partA_ref, partB_ref, partC_ref, WhT_ref,
                      WaT_ref, bn1_ref, Wn2T_ref, bn2_ref, g_ref, bln_ref)
    h2_ref[...] = h2
    pw = jnp.dot(ppos_ref[...], Wr8T_ref[...], preferred_element_type=jnp.float32)
    hA = jnp.dot(h2, WdT_ref[...], preferred_element_type=jnp.float32)
    hB = jnp.dot(h2, WsT_ref[...], preferred_element_type=jnp.float32)
    A_ref[...] = _pack_bf16(hA + pw)
    B_ref[...] = _pack_bf16(hB - pw)


def _tc_node(h, partA, partB, partC, ppos, WhT, WaT, bn1, Wn2T, bn2, g, bln,
             WdT, WsT, Wr8T):
    full = lambda s: pl.BlockSpec(s, lambda i: (0,) * len(s))
    return pl.pallas_call(
        _node_kernel,
        grid=(V // BV,),
        in_specs=[pl.BlockSpec((BV, H), lambda i: (i, 0)),
                  pl.BlockSpec((2, BV, H), lambda i: (0, i, 0)),
                  pl.BlockSpec((2, BV, H), lambda i: (0, i, 0)),
                  pl.BlockSpec((2, BV, H), lambda i: (0, i, 0)),
                  pl.BlockSpec((BV, 8), lambda i: (i, 0)),
                  full((H, H)), full((H, H)), full((1, H)),
                  full((H, H)), full((1, H)), full((1, H)), full((1, H)),
                  full((H, H)), full((H, H)), full((8, H))],
        out_specs=[pl.BlockSpec((BV, H), lambda i: (i, 0)),
                   pl.BlockSpec((BV, H // 2), lambda i: (i, 0)),
                   pl.BlockSpec((BV, H // 2), lambda i: (i, 0))],
        out_shape=[jax.ShapeDtypeStruct((V, H), jnp.float32),
                   jax.ShapeDtypeStruct((V, H // 2), jnp.int32),
                   jax.ShapeDtypeStruct((V, H // 2), jnp.int32)],
    )(h, partA, partB, partC, ppos, WhT, WaT, bn1, Wn2T, bn2, g, bln,
      WdT, WsT, Wr8T)


def _node_final_kernel(h_ref, partA_ref, partB_ref, partC_ref, WhT_ref,
                       WaT_ref, bn1_ref, Wn2T_ref, bn2_ref, g_ref, bln_ref,
                       WoT_ref, bo_ref, out_ref):
    h2 = _node_common(h_ref, partA_ref, partB_ref, partC_ref, WhT_ref,
                      WaT_ref, bn1_ref, Wn2T_ref, bn2_ref, g_ref, bln_ref)
    out_ref[...] = jnp.dot(h2, WoT_ref[...], preferred_element_type=jnp.float32) + bo_ref[...]


def _tc_node_final(h, partA, partB, partC, WhT, WaT, bn1, Wn2T, bn2, g, bln,
                   WoT8, bo8):
    full = lambda s: pl.BlockSpec(s, lambda i: (0,) * len(s))
    return pl.pallas_call(
        _node_final_kernel,
        grid=(V // BV,),
        in_specs=[pl.BlockSpec((BV, H), lambda i: (i, 0)),
                  pl.BlockSpec((2, BV, H), lambda i: (0, i, 0)),
                  pl.BlockSpec((2, BV, H), lambda i: (0, i, 0)),
                  pl.BlockSpec((2, BV, H), lambda i: (0, i, 0)),
                  full((H, H)), full((H, H)), full((1, H)),
                  full((H, H)), full((1, H)), full((1, H)), full((1, H)),
                  full((H, 8)), full((1, 8))],
        out_specs=pl.BlockSpec((BV, 8), lambda i: (i, 0)),
        out_shape=jax.ShapeDtypeStruct((V, 8), jnp.float32),
    )(h, partA, partB, partC, WhT, WaT, bn1, Wn2T, bn2, g, bln, WoT8, bo8)


def kernel(pos, t, z, params, edge_index, batch):
    f32 = jnp.float32
    src = edge_index[0]
    dst = edge_index[1]
    splits, off = [], 0
    for Ex in ESPLIT:
        splits.append((src[off:off + Ex].reshape(NW, Ex // NW // C, C),
                       dst[off:off + Ex].reshape(NW, Ex // NW // C, C), Ex))
        off += Ex

    te0, te1 = params["time_embed"]
    temb = _silu(t[:1, None] * te0["W"][:, 0][None, :] + te0["b"][None, :])
    temb = temb @ te1["W"].T + te1["b"][None, :]

    cw = params["cond_proj"]["W"]
    cpzT = jnp.asarray(cw[:, :64].T, f32)
    cptT = jnp.asarray(cw[:, 64:].T, f32)
    bc = params["cond_proj"]["b"][None, :]

    ppos = jnp.pad(pos, ((0, 0), (0, 5)))

    def split_w1(lp):
        W1 = lp["edge_mlp"][0]["W"]
        WdT = W1[:, :H].T
        WsT = W1[:, H:2 * H].T
        Wr8T = jnp.pad(W1[:, 2 * H:].T, ((0, 5), (0, 0)))
        return WdT, WsT, Wr8T

    lps = params["layers"]
    WdT0, WsT0, Wr8T0 = split_w1(lps[0])
    h, A, B = _tc_prep(z, temb, cpzT, cptT, bc, WdT0, WsT0, Wr8T0, ppos)

    for li, lp in enumerate(lps):
        b1 = lp["edge_mlp"][0]["b"][None, :]
        W2T = lp["edge_mlp"][1]["W"].T
        b2 = lp["edge_mlp"][1]["b"][None, :]
        Gs = [_sc_gather(A, B, d3, s3, Ex) for s3, d3, Ex in splits]
        e2s = [_tc_edge(G, b1, W2T, b2, Ex)
               for G, (_, _, Ex) in zip(Gs, splits)]
        parts = [_sc_scatter(e2, d3, Ex)
                 for e2, (_, d3, Ex) in zip(e2s, splits)]
        partA, partB, partC = parts

        n1, n2 = lp["node_mlp"]
        WhT = n1["W"][:, :H].T
        WaT = n1["W"][:, H:].T
        bn1 = n1["b"][None, :]
        Wn2T = n2["W"].T
        bn2 = n2["b"][None, :]
        g = lp["ln"]["g"][None, :]
        bln = lp["ln"]["b"][None, :]
        if li + 1 < len(lps):
            WdT, WsT, Wr8T = split_w1(lps[li + 1])
            h, A, B = _tc_node(h, partA, partB, partC, ppos, WhT, WaT, bn1,
                               Wn2T, bn2, g, bln, WdT, WsT, Wr8T)
        else:
            WoT8 = jnp.pad(params["out_proj"]["W"].T, ((0, 0), (0, 5)))
            bo8 = jnp.pad(params["out_proj"]["b"], (0, 5))[None, :]
            out8 = _tc_node_final(h, partA, partB, partC, WhT, WaT, bn1, Wn2T,
                                  bn2, g, bln, WoT8, bo8)
    return out8[:, :3]
